# baseline (speedup 1.0000x reference)
import functools

import jax
import jax.numpy as jnp
from jax import lax
from jax.experimental import pallas as pl
from jax.experimental.pallas import tpu as pltpu

NUM_CLASSES = 4
MARGIN = 1.0


def _tc_body(f_ref, t_ref, sums_ref, counts_ref, loss_ref):
    b = pl.program_id(0)
    n = pl.program_id(1)
    nb = pl.num_programs(1)

    @pl.when(jnp.logical_and(b == 0, n == 0))
    def _init():
        sums_ref[...] = jnp.zeros_like(sums_ref)
        counts_ref[...] = jnp.zeros_like(counts_ref)

    f = f_ref[0]
    t = t_ref[0, 0]
    nb_len = f.shape[1]
    cls = lax.broadcasted_iota(jnp.int32, (nb_len, NUM_CLASSES), 1)
    oh = (cls == t.reshape(nb_len, 1)).astype(jnp.float32)
    partial = lax.dot_general(
        f, oh, (((1,), (0,)), ((), ())), preferred_element_type=jnp.float32
    )
    sums_ref[b] += partial
    counts_ref[b] += jnp.sum(oh, axis=0, keepdims=True)

    @pl.when(jnp.logical_and(b == pl.num_programs(0) - 1, n == nb - 1))
    def _finalize():
        sums = sums_ref[...]
        counts = counts_ref[...]
        centers = sums / jnp.maximum(counts, 1.0)
        valid = counts[:, 0, :] > 0
        total = jnp.float32(0.0)
        pairs = jnp.float32(0.0)
        for i in range(NUM_CLASSES):
            for j in range(i + 1, NUM_CLASSES):
                diff = centers[:, :, i] - centers[:, :, j]
                dist = jnp.sqrt(jnp.sum(diff * diff, axis=1))
                hinge = jnp.maximum(MARGIN - dist, 0.0)
                m = jnp.logical_and(valid[:, i], valid[:, j]).astype(jnp.float32)
                total = total + jnp.sum(hinge * m)
                pairs = pairs + jnp.sum(m)
        val = jnp.where(pairs > 0, total / jnp.maximum(pairs, 1.0), 0.0)
        loss_ref[...] = val.reshape(1, 1)


def kernel(features, predictions, targets):
    del predictions
    B, C, H, W, D = features.shape
    N = H * W * D
    feats = features.reshape(B, C, N)
    tgt = targets.reshape(B, 1, N)

    NBLK = 8
    Nb = N // NBLK

    sums, counts, loss = pl.pallas_call(
        _tc_body,
        grid=(B, NBLK),
        in_specs=[
            pl.BlockSpec((1, C, Nb), lambda b, n: (b, 0, n)),
            pl.BlockSpec((1, 1, Nb), lambda b, n: (b, 0, n)),
        ],
        out_specs=[
            pl.BlockSpec((B, C, NUM_CLASSES), lambda b, n: (0, 0, 0)),
            pl.BlockSpec((B, 1, NUM_CLASSES), lambda b, n: (0, 0, 0)),
            pl.BlockSpec((1, 1), lambda b, n: (0, 0)),
        ],
        out_shape=[
            jax.ShapeDtypeStruct((B, C, NUM_CLASSES), jnp.float32),
            jax.ShapeDtypeStruct((B, 1, NUM_CLASSES), jnp.float32),
            jax.ShapeDtypeStruct((1, 1), jnp.float32),
        ],
    )(feats, tgt)
    return loss[0, 0]


# speedup vs baseline: 1.2105x; 1.2105x over previous
import functools

import jax
import jax.numpy as jnp
from jax import lax
from jax.experimental import pallas as pl
from jax.experimental.pallas import tpu as pltpu

NUM_CLASSES = 4
MARGIN = 1.0


def _tc_body(f_ref, t_ref, sums_ref, counts_ref, loss_ref):
    b = pl.program_id(0)
    n = pl.program_id(1)
    nb = pl.num_programs(1)

    @pl.when(jnp.logical_and(b == 0, n == 0))
    def _init():
        sums_ref[...] = jnp.zeros_like(sums_ref)
        counts_ref[...] = jnp.zeros_like(counts_ref)

    f = f_ref[0]
    t = t_ref[0]
    nb_len = f.shape[1]
    s_total = jnp.sum(f, axis=1, keepdims=True)
    zero = jnp.zeros_like(f)
    s_rest = jnp.zeros_like(s_total)
    n_rest = jnp.zeros((1, 1), jnp.float32)
    for k in range(1, NUM_CLASSES):
        m = t == k
        s_k = jnp.sum(jnp.where(m, f, zero), axis=1, keepdims=True)
        n_k = jnp.sum(m.astype(jnp.float32), axis=1, keepdims=True)
        sums_ref[b, :, k : k + 1] += s_k
        counts_ref[b, :, k : k + 1] += n_k
        s_rest = s_rest + s_k
        n_rest = n_rest + n_k
    sums_ref[b, :, 0:1] += s_total - s_rest
    counts_ref[b, :, 0:1] += jnp.float32(nb_len) - n_rest

    @pl.when(jnp.logical_and(b == pl.num_programs(0) - 1, n == nb - 1))
    def _finalize():
        sums = sums_ref[...]
        counts = counts_ref[...]
        centers = sums / jnp.maximum(counts, 1.0)
        valid = counts[:, 0, :] > 0
        total = jnp.float32(0.0)
        pairs = jnp.float32(0.0)
        for i in range(NUM_CLASSES):
            for j in range(i + 1, NUM_CLASSES):
                diff = centers[:, :, i] - centers[:, :, j]
                dist = jnp.sqrt(jnp.sum(diff * diff, axis=1))
                hinge = jnp.maximum(MARGIN - dist, 0.0)
                m = jnp.logical_and(valid[:, i], valid[:, j]).astype(jnp.float32)
                total = total + jnp.sum(hinge * m)
                pairs = pairs + jnp.sum(m)
        val = jnp.where(pairs > 0, total / jnp.maximum(pairs, 1.0), 0.0)
        loss_ref[...] = val.reshape(1, 1)


def kernel(features, predictions, targets):
    del predictions
    B, C, H, W, D = features.shape
    N = H * W * D
    feats = features.reshape(B, C, N)
    tgt = targets.reshape(B, 1, N)

    NBLK = 8
    Nb = N // NBLK

    sums, counts, loss = pl.pallas_call(
        _tc_body,
        grid=(B, NBLK),
        in_specs=[
            pl.BlockSpec((1, C, Nb), lambda b, n: (b, 0, n)),
            pl.BlockSpec((1, 1, Nb), lambda b, n: (b, 0, n)),
        ],
        out_specs=[
            pl.BlockSpec((B, C, NUM_CLASSES), lambda b, n: (0, 0, 0)),
            pl.BlockSpec((B, 1, NUM_CLASSES), lambda b, n: (0, 0, 0)),
            pl.BlockSpec((1, 1), lambda b, n: (0, 0)),
        ],
        out_shape=[
            jax.ShapeDtypeStruct((B, C, NUM_CLASSES), jnp.float32),
            jax.ShapeDtypeStruct((B, 1, NUM_CLASSES), jnp.float32),
            jax.ShapeDtypeStruct((1, 1), jnp.float32),
        ],
    )(feats, tgt)
    return loss[0, 0]
